# baseline (speedup 1.0000x reference)
import jax
import jax.numpy as jnp
from jax.experimental import pallas as pl
from jax.experimental.pallas import tpu as pltpu

BS, L, DIM = 4, 512, 128
IG = 256
NG = L // IG


def _gcn_body(q_ref, qg_ref, qT_ref, qgT_ref, depT_ref, a_ref, b_ref,
              w2bd_ref, wdT_ref, outT_ref):
    prei = jnp.dot(qg_ref[0], a_ref[...],
                   preferred_element_type=jnp.float32)
    prej = jnp.dot(q_ref[0], b_ref[...],
                   preferred_element_type=jnp.float32)

    prej_h = prej.astype(jnp.bfloat16)
    prei_h = prei.astype(jnp.bfloat16)
    zero_h = jnp.bfloat16(0.0)
    pieces = [
        jnp.maximum(prej_h + prei_h[u:u + 1, :], zero_h)
        for u in range(IG)
    ]
    tT = jnp.zeros((L, IG), jnp.float32)
    for g2 in range(IG // 2):
        part = jnp.concatenate(pieces[2 * g2:2 * g2 + 2], axis=1)
        w2s = w2bd_ref[2 * DIM * g2:2 * DIM * (g2 + 1), :]
        tT = tT + jnp.dot(part, w2s, preferred_element_type=jnp.float32)

    maskT = depT_ref[0] > 0
    T = jnp.where(maskT, tT, jnp.float32(-100.0))
    m = jnp.max(T, axis=0, keepdims=True)
    e = jnp.exp(T - m)
    betam = e / jnp.sum(e, axis=0, keepdims=True) * maskT.astype(jnp.float32)

    HjT = jnp.dot(wdT_ref[...], qT_ref[0],
                  preferred_element_type=jnp.float32)
    aggT = jnp.dot(HjT, betam, preferred_element_type=jnp.float32)
    outT_ref[0] = jnp.maximum(qgT_ref[0] + aggT, 0.0)


def kernel(queries, wordlens, syntactic_dep, W1, W2, Wd):
    q = queries.astype(jnp.float32)
    qT = jnp.swapaxes(q, 1, 2)
    depT = jnp.swapaxes(syntactic_dep.astype(jnp.int32), 1, 2)
    A = W1[:DIM, :]
    B = W1[DIM:, :]
    W2bd = jnp.kron(jnp.eye(IG, dtype=jnp.float32),
                    W2).astype(jnp.bfloat16)
    WdT = jnp.swapaxes(Wd, 0, 1)

    outT = pl.pallas_call(
        _gcn_body,
        grid=(BS, NG),
        in_specs=[
            pl.BlockSpec((1, L, DIM), lambda b, g: (b, 0, 0)),
            pl.BlockSpec((1, IG, DIM), lambda b, g: (b, g, 0)),
            pl.BlockSpec((1, DIM, L), lambda b, g: (b, 0, 0)),
            pl.BlockSpec((1, DIM, IG), lambda b, g: (b, 0, g)),
            pl.BlockSpec((1, L, IG), lambda b, g: (b, 0, g)),
            pl.BlockSpec((DIM, DIM), lambda b, g: (0, 0)),
            pl.BlockSpec((DIM, DIM), lambda b, g: (0, 0)),
            pl.BlockSpec((IG * DIM, IG), lambda b, g: (0, 0)),
            pl.BlockSpec((DIM, DIM), lambda b, g: (0, 0)),
        ],
        out_specs=pl.BlockSpec((1, DIM, IG), lambda b, g: (b, 0, g)),
        out_shape=jax.ShapeDtypeStruct((BS, DIM, L), jnp.float32),
        compiler_params=pltpu.CompilerParams(
            dimension_semantics=("arbitrary", "arbitrary"),
        ),
    )(q, q, qT, qT, depT, A, B, W2bd, WdT)

    out = jnp.swapaxes(outT, 1, 2)
    return (out, wordlens, syntactic_dep)


# speedup vs baseline: 1.0718x; 1.0718x over previous
import jax
import jax.numpy as jnp
from jax.experimental import pallas as pl
from jax.experimental.pallas import tpu as pltpu

BS, L, DIM = 4, 512, 128
IG = 128
NG = L // IG


def _gcn_body(q_ref, qg_ref, qT_ref, qgT_ref, depT_ref, a_ref, b_ref,
              w2bd_ref, wdT_ref, outT_ref):
    prei = jnp.dot(qg_ref[0], a_ref[...],
                   preferred_element_type=jnp.float32)
    prej = jnp.dot(q_ref[0], b_ref[...],
                   preferred_element_type=jnp.float32)

    prej_h = prej.astype(jnp.bfloat16)
    prei_h = prei.astype(jnp.bfloat16)
    zero_h = jnp.bfloat16(0.0)
    pieces = [
        jnp.maximum(prej_h + prei_h[u:u + 1, :], zero_h)
        for u in range(IG)
    ]
    tT = jnp.zeros((L, IG), jnp.float32)
    for g2 in range(IG // 2):
        part = jnp.concatenate(pieces[2 * g2:2 * g2 + 2], axis=1)
        w2s = w2bd_ref[2 * DIM * g2:2 * DIM * (g2 + 1), :]
        tT = tT + jnp.dot(part, w2s, preferred_element_type=jnp.float32)

    maskT = depT_ref[0] > 0
    T = jnp.where(maskT, tT, jnp.float32(-100.0))
    m = jnp.max(T, axis=0, keepdims=True)
    e = jnp.exp(T - m)
    betam = e / jnp.sum(e, axis=0, keepdims=True) * maskT.astype(jnp.float32)

    HjT = jnp.dot(wdT_ref[...], qT_ref[0],
                  preferred_element_type=jnp.float32)
    aggT = jnp.dot(HjT, betam, preferred_element_type=jnp.float32)
    outT_ref[0] = jnp.maximum(qgT_ref[0] + aggT, 0.0)


def kernel(queries, wordlens, syntactic_dep, W1, W2, Wd):
    q = queries.astype(jnp.float32)
    qT = jnp.swapaxes(q, 1, 2)
    depT = jnp.swapaxes(syntactic_dep.astype(jnp.int32), 1, 2)
    A = W1[:DIM, :]
    B = W1[DIM:, :]
    W2bd = jnp.kron(jnp.eye(IG, dtype=jnp.float32),
                    W2).astype(jnp.bfloat16)
    WdT = jnp.swapaxes(Wd, 0, 1)

    outT = pl.pallas_call(
        _gcn_body,
        grid=(BS, NG),
        in_specs=[
            pl.BlockSpec((1, L, DIM), lambda b, g: (b, 0, 0)),
            pl.BlockSpec((1, IG, DIM), lambda b, g: (b, g, 0)),
            pl.BlockSpec((1, DIM, L), lambda b, g: (b, 0, 0)),
            pl.BlockSpec((1, DIM, IG), lambda b, g: (b, 0, g)),
            pl.BlockSpec((1, L, IG), lambda b, g: (b, 0, g)),
            pl.BlockSpec((DIM, DIM), lambda b, g: (0, 0)),
            pl.BlockSpec((DIM, DIM), lambda b, g: (0, 0)),
            pl.BlockSpec((IG * DIM, IG), lambda b, g: (0, 0)),
            pl.BlockSpec((DIM, DIM), lambda b, g: (0, 0)),
        ],
        out_specs=pl.BlockSpec((1, DIM, IG), lambda b, g: (b, 0, g)),
        out_shape=jax.ShapeDtypeStruct((BS, DIM, L), jnp.float32),
        compiler_params=pltpu.CompilerParams(
            dimension_semantics=("arbitrary", "arbitrary"),
        ),
    )(q, q, qT, qT, depT, A, B, W2bd, WdT)

    out = jnp.swapaxes(outT, 1, 2)
    return (out, wordlens, syntactic_dep)
